# baseline (speedup 1.0000x reference)
import functools

import jax
import jax.numpy as jnp
from jax import lax
from jax.experimental import pallas as pl
from jax.experimental.pallas import tpu as pltpu
from jax.experimental.pallas import tpu_sc as plsc

_R = 8192
_B = 64
_M = 16
_D = 128
_S = 16
_NEG = -1e30


def _z():
    return jnp.int32(0)


def _f(v):
    return jnp.float32(v)


def _stage1_body(batch_ref, x_ref, w_ref, b_ref, u_ref,
                 scores_ref, t_ref,
                 m_ref, s_ref, cnt_ref):
    i = pl.program_id(0)
    nb = pl.num_programs(0)

    @pl.when(i == 0)
    def _init():
        m_ref[...] = jnp.full_like(m_ref, _f(_NEG))
        s_ref[...] = jnp.zeros_like(s_ref)
        cnt_ref[...] = jnp.zeros_like(cnt_ref)

    x = x_ref[...]
    h = lax.dot_general(x, w_ref[...], (((1,), (1,)), ((), ())),
                        preferred_element_type=jnp.float32)
    h = h + b_ref[...]
    h = _f(0.5) * h * (_f(1.0) + lax.erf(h * _f(0.7071067811865476)))

    bl = batch_ref[0]
    b_lo = jnp.min(bl)
    b_hi = jnp.minimum(jnp.max(bl), jnp.int32(_B - 1))

    b_lo_s = jnp.minimum(b_lo, jnp.int32(_B - _S))
    u_blk = u_ref[pl.ds(b_lo_s, _S)].reshape(_S * _M, _D)
    contrib_all = lax.dot_general(u_blk, h, (((1,), (1,)), ((), ())),
                                  preferred_element_type=jnp.float32)
    jr = bl - b_lo_s
    scoresT = jnp.zeros((_M, _R), jnp.float32)
    for j in range(_S):
        scoresT = jnp.where(jr == j, contrib_all[j * _M:(j + 1) * _M, :],
                            scoresT)

    def assemble(g, acc):
        u_g = u_ref[g]
        contribT = lax.dot_general(u_g, h, (((1,), (1,)), ((), ())),
                                   preferred_element_type=jnp.float32)
        return jnp.where(bl == g, contribT, acc)

    scoresT = lax.fori_loop(b_lo_s + jnp.int32(_S), b_hi + jnp.int32(1),
                            assemble, scoresT)
    scores_ref[...] = scoresT

    c_blk = jnp.max(scoresT, axis=1, keepdims=True)
    e = jnp.exp(scoresT - c_blk)

    iota_b = lax.broadcasted_iota(jnp.int32, (_B, 1), 0)
    ohF = (bl == iota_b).astype(jnp.float32)
    b_sum = lax.dot_general(e, ohF,
                            (((1,), (1,)), ((), ())),
                            preferred_element_type=jnp.float32)
    b_cnt = lax.dot_general(jnp.ones((1, _R), jnp.float32), ohF,
                            (((1,), (1,)), ((), ())),
                            preferred_element_type=jnp.float32)
    present = b_cnt > _f(0.0)
    m_old = m_ref[...]
    m_new = jnp.where(present, jnp.maximum(m_old, c_blk), m_old)
    scale_new = jnp.where(present, jnp.exp(c_blk - m_new), _f(0.0))
    s_ref[...] = s_ref[...] * jnp.exp(m_old - m_new) + b_sum * scale_new
    m_ref[...] = m_new
    cnt_ref[...] = cnt_ref[...] + b_cnt

    @pl.when(i == nb - 1)
    def _finalize():
        cnt = cnt_ref[...]
        n_max = jnp.max(cnt)
        pad = n_max - cnt
        m = m_ref[...]
        r = jnp.where(pad > _f(0.0), jnp.maximum(m, _f(0.0)), m)
        denom = s_ref[...] * jnp.exp(m - r) + pad * jnp.exp(-r)
        t = r + jnp.log(denom)
        t_ref[...] = jnp.concatenate(
            [t, jnp.full((_M, 1), _f(1e30), jnp.float32)], axis=1)


@functools.lru_cache(maxsize=None)
def _make_stage1(n_pad):
    nb = n_pad // _R
    return pl.pallas_call(
        _stage1_body,
        grid=(nb,),
        in_specs=[
            pl.BlockSpec((1, 1, _R), lambda i: (i, _z(), _z())),
            pl.BlockSpec((_R, _D), lambda i: (i, _z())),
            pl.BlockSpec((_D, _D), lambda i: (_z(), _z())),
            pl.BlockSpec((1, _D), lambda i: (_z(), _z())),
            pl.BlockSpec((_B, _M, _D),
                         lambda i: (_z(), _z(), _z())),
        ],
        out_specs=[
            pl.BlockSpec((_M, _R), lambda i: (_z(), i)),
            pl.BlockSpec((_M, _B + 1), lambda i: (_z(), _z())),
        ],
        out_shape=[
            jax.ShapeDtypeStruct((_M, n_pad), jnp.float32),
            jax.ShapeDtypeStruct((_M, _B + 1), jnp.float32),
        ],
        scratch_shapes=[
            pltpu.VMEM((_M, _B), jnp.float32),
            pltpu.VMEM((_M, _B), jnp.float32),
            pltpu.VMEM((1, _B), jnp.float32),
        ],
        compiler_params=pltpu.CompilerParams(
            dimension_semantics=("arbitrary",)),
    )


@functools.lru_cache(maxsize=None)
def _make_stage2(n_pad):
    n_workers = 32
    ch = n_pad // n_workers
    stats = _M * (_B + 1)
    mesh = plsc.VectorSubcoreMesh(core_axis_name="c", subcore_axis_name="s")

    def body(scores_hbm, batch_hbm, t_hbm, out_hbm,
             sc_v, b_v, t_v, o_v):
        wid = lax.axis_index("c") * 16 + lax.axis_index("s")
        base = wid * ch
        pltpu.sync_copy(scores_hbm.at[:, pl.ds(base, ch)], sc_v)
        pltpu.sync_copy(batch_hbm.at[pl.ds(base, ch)], b_v)
        pltpu.sync_copy(t_hbm, t_v)
        lane = lax.iota(jnp.int32, 16)

        def group(g, carry):
            g = g.astype(jnp.int32)
            node0 = g * jnp.int32(16)
            bvec = b_v[pl.ds(node0, 16)]
            nidx = node0 + lane
            acc = jnp.zeros((16,), jnp.float32)
            for m in range(_M):
                mi = jnp.full((16,), m, jnp.int32)
                sidx = jnp.int32(m * (_B + 1)) + bvec
                s = plsc.load_gather(sc_v, [mi, nidx])
                t = plsc.load_gather(t_v, [sidx])
                acc = acc + jnp.exp(s - t)
            one = _f(1.0)
            sig = one / (one + jnp.exp(-acc))
            o_v[pl.ds(node0, 16)] = jnp.maximum(
                jnp.sign(sig - _f(0.5)), _f(0.0))
            return carry

        lax.fori_loop(jnp.int32(0), jnp.int32(ch // 16), group, jnp.int32(0))
        pltpu.sync_copy(o_v, out_hbm.at[pl.ds(base, ch)])

    return pl.kernel(
        body,
        mesh=mesh,
        compiler_params=pltpu.CompilerParams(needs_layout_passes=False),
        out_type=jax.ShapeDtypeStruct((n_pad,), jnp.float32),
        scratch_types=[
            pltpu.VMEM((_M, ch), jnp.float32),
            pltpu.VMEM((ch,), jnp.int32),
            pltpu.VMEM((stats,), jnp.float32),
            pltpu.VMEM((ch,), jnp.float32),
        ],
    )


def kernel(x, u, batch, edge_index, W1, b1):
    n = x.shape[0]
    align = max(_R, 4096)
    n_pad = -(-n // align) * align
    b32 = jnp.pad(batch.astype(jnp.int32), (0, n_pad - n),
                  constant_values=_B)
    batch3 = b32.reshape(n_pad // _R, 1, _R)
    u_bmd = jnp.transpose(u.astype(jnp.float32), (1, 0, 2))
    scoresT, tstat = _make_stage1(n_pad)(
        batch3, x.astype(jnp.float32), W1.astype(jnp.float32),
        b1.astype(jnp.float32).reshape(1, _D), u_bmd)
    gate = _make_stage2(n_pad)(scoresT, b32, tstat.reshape(-1))
    return gate[:n]


# speedup vs baseline: 1.1352x; 1.1352x over previous
import functools

import jax
import jax.numpy as jnp
from jax import lax
from jax.experimental import pallas as pl
from jax.experimental.pallas import tpu as pltpu
from jax.experimental.pallas import tpu_sc as plsc

_R = 8192
_B = 64
_M = 16
_D = 128
_S = 8
_NEG = -1e30


def _z():
    return jnp.int32(0)


def _f(v):
    return jnp.float32(v)


def _stage1_body(batch_ref, x_ref, w_ref, b_ref, u_ref,
                 scores_ref, t_ref,
                 m_ref, s_ref, cnt_ref):
    i = pl.program_id(0)
    nb = pl.num_programs(0)

    @pl.when(i == 0)
    def _init():
        m_ref[...] = jnp.full_like(m_ref, _f(_NEG))
        s_ref[...] = jnp.zeros_like(s_ref)
        cnt_ref[...] = jnp.zeros_like(cnt_ref)

    x = x_ref[...]
    h = lax.dot_general(x, w_ref[...], (((1,), (1,)), ((), ())),
                        preferred_element_type=jnp.float32)
    h = h + b_ref[...]
    h = _f(0.5) * h * (_f(1.0) + lax.erf(h * _f(0.7071067811865476)))

    bl = batch_ref[0]
    b_lo = jnp.min(bl)
    b_hi = jnp.minimum(jnp.max(bl), jnp.int32(_B - 1))

    b_lo_s = jnp.minimum(b_lo, jnp.int32(_B - _S))
    u_blk = u_ref[pl.ds(b_lo_s, _S)].reshape(_S * _M, _D)
    contrib_all = lax.dot_general(u_blk, h, (((1,), (1,)), ((), ())),
                                  preferred_element_type=jnp.float32)
    jr = bl - b_lo_s
    scoresT = jnp.zeros((_M, _R), jnp.float32)
    for j in range(_S):
        scoresT = jnp.where(jr == j, contrib_all[j * _M:(j + 1) * _M, :],
                            scoresT)

    def assemble(g, acc):
        u_g = u_ref[g]
        contribT = lax.dot_general(u_g, h, (((1,), (1,)), ((), ())),
                                   preferred_element_type=jnp.float32)
        return jnp.where(bl == g, contribT, acc)

    scoresT = lax.fori_loop(b_lo_s + jnp.int32(_S), b_hi + jnp.int32(1),
                            assemble, scoresT)
    scores_ref[...] = scoresT

    c_blk = jnp.max(scoresT, axis=1, keepdims=True)
    e = jnp.exp(scoresT - c_blk)

    iota_b = lax.broadcasted_iota(jnp.int32, (_B, 1), 0)
    ohF = (bl == iota_b).astype(jnp.float32)
    b_sum = lax.dot_general(e, ohF,
                            (((1,), (1,)), ((), ())),
                            preferred_element_type=jnp.float32)
    b_cnt = lax.dot_general(jnp.ones((1, _R), jnp.float32), ohF,
                            (((1,), (1,)), ((), ())),
                            preferred_element_type=jnp.float32)
    present = b_cnt > _f(0.0)
    m_old = m_ref[...]
    m_new = jnp.where(present, jnp.maximum(m_old, c_blk), m_old)
    scale_new = jnp.where(present, jnp.exp(c_blk - m_new), _f(0.0))
    s_ref[...] = s_ref[...] * jnp.exp(m_old - m_new) + b_sum * scale_new
    m_ref[...] = m_new
    cnt_ref[...] = cnt_ref[...] + b_cnt

    @pl.when(i == nb - 1)
    def _finalize():
        cnt = cnt_ref[...]
        n_max = jnp.max(cnt)
        pad = n_max - cnt
        m = m_ref[...]
        r = jnp.where(pad > _f(0.0), jnp.maximum(m, _f(0.0)), m)
        denom = s_ref[...] * jnp.exp(m - r) + pad * jnp.exp(-r)
        t = r + jnp.log(denom)
        t_ref[...] = jnp.concatenate(
            [t, jnp.full((_M, 1), _f(1e30), jnp.float32)], axis=1)


@functools.lru_cache(maxsize=None)
def _make_stage1(n_pad):
    nb = n_pad // _R
    return pl.pallas_call(
        _stage1_body,
        grid=(nb,),
        in_specs=[
            pl.BlockSpec((1, 1, _R), lambda i: (i, _z(), _z())),
            pl.BlockSpec((_R, _D), lambda i: (i, _z())),
            pl.BlockSpec((_D, _D), lambda i: (_z(), _z())),
            pl.BlockSpec((1, _D), lambda i: (_z(), _z())),
            pl.BlockSpec((_B, _M, _D),
                         lambda i: (_z(), _z(), _z())),
        ],
        out_specs=[
            pl.BlockSpec((_M, _R), lambda i: (_z(), i)),
            pl.BlockSpec((_M, _B + 1), lambda i: (_z(), _z())),
        ],
        out_shape=[
            jax.ShapeDtypeStruct((_M, n_pad), jnp.float32),
            jax.ShapeDtypeStruct((_M, _B + 1), jnp.float32),
        ],
        scratch_shapes=[
            pltpu.VMEM((_M, _B), jnp.float32),
            pltpu.VMEM((_M, _B), jnp.float32),
            pltpu.VMEM((1, _B), jnp.float32),
        ],
        compiler_params=pltpu.CompilerParams(
            dimension_semantics=("arbitrary",)),
    )


@functools.lru_cache(maxsize=None)
def _make_stage2(n_pad):
    n_workers = 32
    ch = n_pad // n_workers
    stats = _M * (_B + 1)
    mesh = plsc.VectorSubcoreMesh(core_axis_name="c", subcore_axis_name="s")

    def body(scores_hbm, batch_hbm, t_hbm, out_hbm,
             sc_v, b_v, t_v, o_v):
        wid = lax.axis_index("c") * 16 + lax.axis_index("s")
        base = wid * ch
        pltpu.sync_copy(scores_hbm.at[:, pl.ds(base, ch)], sc_v)
        pltpu.sync_copy(batch_hbm.at[pl.ds(base, ch)], b_v)
        pltpu.sync_copy(t_hbm, t_v)

        def group(g, carry):
            g = g.astype(jnp.int32)
            node0 = g * jnp.int32(16)
            bvec = b_v[pl.ds(node0, 16)]
            acc = jnp.zeros((16,), jnp.float32)
            for m in range(_M):
                sidx = jnp.int32(m * (_B + 1)) + bvec
                s = sc_v[m, pl.ds(node0, 16)]
                t = plsc.load_gather(t_v, [sidx])
                acc = acc + jnp.exp(s - t)
            one = _f(1.0)
            sig = one / (one + jnp.exp(-acc))
            o_v[pl.ds(node0, 16)] = jnp.maximum(
                jnp.sign(sig - _f(0.5)), _f(0.0))
            return carry

        lax.fori_loop(jnp.int32(0), jnp.int32(ch // 16), group, jnp.int32(0))
        pltpu.sync_copy(o_v, out_hbm.at[pl.ds(base, ch)])

    return pl.kernel(
        body,
        mesh=mesh,
        compiler_params=pltpu.CompilerParams(needs_layout_passes=False),
        out_type=jax.ShapeDtypeStruct((n_pad,), jnp.float32),
        scratch_types=[
            pltpu.VMEM((_M, ch), jnp.float32),
            pltpu.VMEM((ch,), jnp.int32),
            pltpu.VMEM((stats,), jnp.float32),
            pltpu.VMEM((ch,), jnp.float32),
        ],
    )


def kernel(x, u, batch, edge_index, W1, b1):
    n = x.shape[0]
    align = max(_R, 4096)
    n_pad = -(-n // align) * align
    b32 = jnp.pad(batch.astype(jnp.int32), (0, n_pad - n),
                  constant_values=_B)
    batch3 = b32.reshape(n_pad // _R, 1, _R)
    u_bmd = jnp.transpose(u.astype(jnp.float32), (1, 0, 2))
    scoresT, tstat = _make_stage1(n_pad)(
        batch3, x.astype(jnp.float32), W1.astype(jnp.float32),
        b1.astype(jnp.float32).reshape(1, _D), u_bmd)
    gate = _make_stage2(n_pad)(scoresT, b32, tstat.reshape(-1))
    return gate[:n]


# speedup vs baseline: 1.1407x; 1.0049x over previous
import functools

import jax
import jax.numpy as jnp
from jax import lax
from jax.experimental import pallas as pl
from jax.experimental.pallas import tpu as pltpu
from jax.experimental.pallas import tpu_sc as plsc

_R = 8192
_B = 64
_M = 16
_D = 128
_S = 8
_NEG = -1e30


def _z():
    return jnp.int32(0)


def _f(v):
    return jnp.float32(v)


def _stage1_body(batch_ref, x_ref, w_ref, b_ref, u_ref,
                 scores_ref, t_ref,
                 m_ref, s_ref, cnt_ref):
    i = pl.program_id(0)
    nb = pl.num_programs(0)

    @pl.when(i == 0)
    def _init():
        m_ref[...] = jnp.full_like(m_ref, _f(_NEG))
        s_ref[...] = jnp.zeros_like(s_ref)
        cnt_ref[...] = jnp.zeros_like(cnt_ref)

    x = x_ref[...]
    h = lax.dot_general(x, w_ref[...], (((1,), (1,)), ((), ())),
                        preferred_element_type=jnp.float32)
    h = h + b_ref[...]
    h = _f(0.5) * h * (_f(1.0) + lax.erf(h * _f(0.7071067811865476)))

    bl = batch_ref[0]
    b_lo = jnp.min(bl)
    b_hi = jnp.minimum(jnp.max(bl), jnp.int32(_B - 1))

    b_lo_s = jnp.minimum(b_lo, jnp.int32(_B - _S))
    u_blk = u_ref[pl.ds(b_lo_s, _S)].reshape(_S * _M, _D)
    contrib_all = lax.dot_general(u_blk, h, (((1,), (1,)), ((), ())),
                                  preferred_element_type=jnp.float32)
    jr = bl - b_lo_s
    scoresT = jnp.zeros((_M, _R), jnp.float32)
    for j in range(_S):
        scoresT = jnp.where(jr == j, contrib_all[j * _M:(j + 1) * _M, :],
                            scoresT)

    def assemble(g, acc):
        u_g = u_ref[g]
        contribT = lax.dot_general(u_g, h, (((1,), (1,)), ((), ())),
                                   preferred_element_type=jnp.float32)
        return jnp.where(bl == g, contribT, acc)

    scoresT = lax.fori_loop(b_lo_s + jnp.int32(_S), b_hi + jnp.int32(1),
                            assemble, scoresT)
    scores_ref[...] = scoresT

    c_blk = jnp.max(scoresT, axis=1, keepdims=True)
    e = jnp.exp(scoresT - c_blk)

    iota_b = lax.broadcasted_iota(jnp.int32, (_B, 1), 0)
    ohF = (bl == iota_b).astype(jnp.float32)
    b_sum = lax.dot_general(e, ohF,
                            (((1,), (1,)), ((), ())),
                            preferred_element_type=jnp.float32)
    b_cnt = lax.dot_general(jnp.ones((1, _R), jnp.float32), ohF,
                            (((1,), (1,)), ((), ())),
                            preferred_element_type=jnp.float32)
    present = b_cnt > _f(0.0)
    m_old = m_ref[...]
    m_new = jnp.where(present, jnp.maximum(m_old, c_blk), m_old)
    scale_new = jnp.where(present, jnp.exp(c_blk - m_new), _f(0.0))
    s_ref[...] = s_ref[...] * jnp.exp(m_old - m_new) + b_sum * scale_new
    m_ref[...] = m_new
    cnt_ref[...] = cnt_ref[...] + b_cnt

    @pl.when(i == nb - 1)
    def _finalize():
        cnt = cnt_ref[...]
        n_max = jnp.max(cnt)
        pad = n_max - cnt
        m = m_ref[...]
        r = jnp.where(pad > _f(0.0), jnp.maximum(m, _f(0.0)), m)
        denom = s_ref[...] * jnp.exp(m - r) + pad * jnp.exp(-r)
        t = r + jnp.log(denom)
        t_ref[...] = jnp.concatenate(
            [t, jnp.full((_M, 1), _f(1e30), jnp.float32)], axis=1)


@functools.lru_cache(maxsize=None)
def _make_stage1(n_pad):
    nb = n_pad // _R
    return pl.pallas_call(
        _stage1_body,
        grid=(nb,),
        in_specs=[
            pl.BlockSpec((1, 1, _R), lambda i: (i, _z(), _z())),
            pl.BlockSpec((_R, _D), lambda i: (i, _z())),
            pl.BlockSpec((_D, _D), lambda i: (_z(), _z())),
            pl.BlockSpec((1, _D), lambda i: (_z(), _z())),
            pl.BlockSpec((_B, _M, _D),
                         lambda i: (_z(), _z(), _z())),
        ],
        out_specs=[
            pl.BlockSpec((_M, _R), lambda i: (_z(), i)),
            pl.BlockSpec((_M, _B + 1), lambda i: (_z(), _z())),
        ],
        out_shape=[
            jax.ShapeDtypeStruct((_M, n_pad), jnp.float32),
            jax.ShapeDtypeStruct((_M, _B + 1), jnp.float32),
        ],
        scratch_shapes=[
            pltpu.VMEM((_M, _B), jnp.float32),
            pltpu.VMEM((_M, _B), jnp.float32),
            pltpu.VMEM((1, _B), jnp.float32),
        ],
        compiler_params=pltpu.CompilerParams(
            dimension_semantics=("arbitrary",)),
    )


@functools.lru_cache(maxsize=None)
def _make_stage2(n_pad):
    n_workers = 32
    ch = n_pad // n_workers
    stats = _M * (_B + 1)
    mesh = plsc.VectorSubcoreMesh(core_axis_name="c", subcore_axis_name="s")

    def body(scores_hbm, batch_hbm, t_hbm, out_hbm,
             sc_v, b_v, t_v, o_v):
        wid = lax.axis_index("c") * 16 + lax.axis_index("s")
        base = wid * ch
        pltpu.sync_copy(scores_hbm.at[:, pl.ds(base, ch)], sc_v)
        pltpu.sync_copy(batch_hbm.at[pl.ds(base, ch)], b_v)
        pltpu.sync_copy(t_hbm, t_v)

        def group(g, carry):
            g = g.astype(jnp.int32)
            node0 = g * jnp.int32(16)
            bvec = b_v[pl.ds(node0, 16)]
            parts = [jnp.zeros((16,), jnp.float32) for _ in range(4)]
            for m in range(_M):
                sidx = jnp.int32(m * (_B + 1)) + bvec
                s = sc_v[m, pl.ds(node0, 16)]
                t = plsc.load_gather(t_v, [sidx])
                parts[m % 4] = parts[m % 4] + jnp.exp(s - t)
            acc = (parts[0] + parts[1]) + (parts[2] + parts[3])
            one = _f(1.0)
            sig = one / (one + jnp.exp(-acc))
            o_v[pl.ds(node0, 16)] = jnp.maximum(
                jnp.sign(sig - _f(0.5)), _f(0.0))
            return carry

        lax.fori_loop(jnp.int32(0), jnp.int32(ch // 16), group, jnp.int32(0))
        pltpu.sync_copy(o_v, out_hbm.at[pl.ds(base, ch)])

    return pl.kernel(
        body,
        mesh=mesh,
        compiler_params=pltpu.CompilerParams(needs_layout_passes=False),
        out_type=jax.ShapeDtypeStruct((n_pad,), jnp.float32),
        scratch_types=[
            pltpu.VMEM((_M, ch), jnp.float32),
            pltpu.VMEM((ch,), jnp.int32),
            pltpu.VMEM((stats,), jnp.float32),
            pltpu.VMEM((ch,), jnp.float32),
        ],
    )


def kernel(x, u, batch, edge_index, W1, b1):
    n = x.shape[0]
    align = max(_R, 4096)
    n_pad = -(-n // align) * align
    b32 = jnp.pad(batch.astype(jnp.int32), (0, n_pad - n),
                  constant_values=_B)
    batch3 = b32.reshape(n_pad // _R, 1, _R)
    u_bmd = jnp.transpose(u.astype(jnp.float32), (1, 0, 2))
    scoresT, tstat = _make_stage1(n_pad)(
        batch3, x.astype(jnp.float32), W1.astype(jnp.float32),
        b1.astype(jnp.float32).reshape(1, _D), u_bmd)
    gate = _make_stage2(n_pad)(scoresT, b32, tstat.reshape(-1))
    return gate[:n]
